# baseline (speedup 1.0000x reference)
import functools

import jax
import jax.numpy as jnp
from jax import lax
from jax.experimental import pallas as pl
from jax.experimental.pallas import tpu as pltpu
from jax.experimental.pallas import tpu_sc as plsc

N_ROWS = 16384
N_FEATS = 512
OUT_F = 64

NC = 2
NS = 16
NW = NC * NS
ROWS_PER_W = N_ROWS // NW
BLK = 64
NBLK = ROWS_PER_W // BLK


def _sc_body(x_hbm, cols_hbm, out_hbm, cols_v, x_v, out_v):
    wid = lax.axis_index("s") * NC + lax.axis_index("c")
    row_base = wid * ROWS_PER_W
    pltpu.sync_copy(cols_hbm, cols_v)

    def do_block(b, _):
        start = row_base + b * BLK
        pltpu.sync_copy(x_hbm.at[pl.ds(start * N_FEATS, BLK * N_FEATS)], x_v)

        def do_row(r, _):
            rbase = r * N_FEATS
            for jb in range(OUT_F // 16):
                cv = cols_v[pl.ds(jb * 16, 16)] + rbase
                g = plsc.load_gather(x_v, [cv])
                out_v[pl.ds(r * OUT_F + jb * 16, 16)] = g
            return 0

        lax.fori_loop(0, BLK, do_row, 0)
        pltpu.sync_copy(out_v, out_hbm.at[pl.ds(start * OUT_F, BLK * OUT_F)])
        return 0

    lax.fori_loop(0, NBLK, do_block, 0)


@jax.jit
def _sc_gather(x_flat, cols):
    mesh = plsc.VectorSubcoreMesh(core_axis_name="c", subcore_axis_name="s")
    return pl.kernel(
        _sc_body,
        out_type=jax.ShapeDtypeStruct((N_ROWS * OUT_F,), jnp.float32),
        mesh=mesh,
        scratch_types=[
            pltpu.VMEM((OUT_F,), jnp.int32),
            pltpu.VMEM((BLK * N_FEATS,), jnp.float32),
            pltpu.VMEM((BLK * OUT_F,), jnp.float32),
        ],
        compiler_params=pltpu.CompilerParams(needs_layout_passes=False),
    )(x_flat, cols)


def kernel(x, columns):
    out = _sc_gather(x.reshape(-1), columns.astype(jnp.int32))
    return out.reshape(N_ROWS, OUT_F)


# speedup vs baseline: 1.2238x; 1.2238x over previous
import jax
import jax.numpy as jnp
from jax import lax
from jax.experimental import pallas as pl
from jax.experimental.pallas import tpu as pltpu
from jax.experimental.pallas import tpu_sc as plsc

N_ROWS = 16384
N_FEATS = 512
OUT_F = 64

NC = 2
NS = 16
NW = NC * NS
ROWS_PER_W = N_ROWS // NW
BLK = 64
NBLK = ROWS_PER_W // BLK
UNROLL = 4


def _sc_body(x_hbm, cols_hbm, out_hbm,
             cols_v, xa, xb, oa, ob, sxa, sxb, soa, sob):
    wid = lax.axis_index("s") * NC + lax.axis_index("c")
    row_base = wid * ROWS_PER_W
    pltpu.sync_copy(cols_hbm, cols_v)

    cbase = tuple(cols_v[pl.ds(j * 16, 16)] for j in range(OUT_F // 16))

    x_bufs = (xa, xb)
    o_bufs = (oa, ob)
    x_sems = (sxa, sxb)
    o_sems = (soa, sob)

    def issue_x(b):
        start = row_base + b * BLK
        return pltpu.async_copy(
            x_hbm.at[pl.ds(start * N_FEATS, BLK * N_FEATS)],
            x_bufs[b % 2], x_sems[b % 2])

    def issue_o(b):
        start = row_base + b * BLK
        return pltpu.async_copy(
            o_bufs[b % 2],
            out_hbm.at[pl.ds(start * OUT_F, BLK * OUT_F)],
            o_sems[b % 2])

    o_descs = {}
    d = issue_x(0)
    for b in range(NBLK):
        d_next = issue_x(b + 1) if b + 1 < NBLK else None
        d.wait()
        if b >= 2:
            o_descs[b - 2].wait()
        x_v = x_bufs[b % 2]
        o_v = o_bufs[b % 2]

        def do_rows(i, cs, x_v=x_v, o_v=o_v):
            off = i * (UNROLL * OUT_F)
            for u in range(UNROLL):
                for jb in range(OUT_F // 16):
                    g = plsc.load_gather(x_v, [cs[jb]])
                    o_v[pl.ds(off + u * OUT_F + jb * 16, 16)] = g
                cs = tuple(c + N_FEATS for c in cs)
            return cs

        lax.fori_loop(0, BLK // UNROLL, do_rows, cbase)
        o_descs[b] = issue_o(b)
        d = d_next

    o_descs[NBLK - 2].wait()
    o_descs[NBLK - 1].wait()


@jax.jit
def _sc_gather(x_flat, cols):
    mesh = plsc.VectorSubcoreMesh(core_axis_name="c", subcore_axis_name="s")
    return pl.kernel(
        _sc_body,
        out_type=jax.ShapeDtypeStruct((N_ROWS * OUT_F,), jnp.float32),
        mesh=mesh,
        scratch_types=[
            pltpu.VMEM((OUT_F,), jnp.int32),
            pltpu.VMEM((BLK * N_FEATS,), jnp.float32),
            pltpu.VMEM((BLK * N_FEATS,), jnp.float32),
            pltpu.VMEM((BLK * OUT_F,), jnp.float32),
            pltpu.VMEM((BLK * OUT_F,), jnp.float32),
            pltpu.SemaphoreType.DMA,
            pltpu.SemaphoreType.DMA,
            pltpu.SemaphoreType.DMA,
            pltpu.SemaphoreType.DMA,
        ],
        compiler_params=pltpu.CompilerParams(needs_layout_passes=False),
    )(x_flat, cols)


def kernel(x, columns):
    out = _sc_gather(x.reshape(-1), columns.astype(jnp.int32))
    return out.reshape(N_ROWS, OUT_F)


# speedup vs baseline: 2.0515x; 1.6764x over previous
import jax
import jax.numpy as jnp
from jax import lax
from jax.experimental import pallas as pl
from jax.experimental.pallas import tpu as pltpu
from jax.experimental.pallas import tpu_sc as plsc

N_ROWS = 16384
N_FEATS = 512
OUT_F = 64

NC = 2
NS = 16
NW = NC * NS
ROWS_PER_W = N_ROWS // NW
BLK = 64
NBLK = ROWS_PER_W // BLK
UNROLL = 4


def _sc_body(x_hbm, cols_hbm, out_hbm,
             cols_v, xa, xb, oa, ob, sxa, sxb, soa, sob):
    wid = lax.axis_index("s") * NC + lax.axis_index("c")
    row_base = wid * ROWS_PER_W
    pltpu.sync_copy(cols_hbm, cols_v)

    cbase = tuple(cols_v[pl.ds(j * 16, 16)] for j in range(OUT_F // 16))
    zeros = jnp.zeros((16,), jnp.int32)

    x_bufs = (xa, xb)
    o_bufs = (oa, ob)
    x_sems = (sxa, sxb)
    o_sems = (soa, sob)

    def issue_x(b):
        start = row_base + b * BLK
        return pltpu.async_copy(
            x_hbm.at[pl.ds(start, BLK)], x_bufs[b % 2], x_sems[b % 2])

    def issue_o(b):
        start = row_base + b * BLK
        return pltpu.async_copy(
            o_bufs[b % 2], out_hbm.at[pl.ds(start, BLK)], o_sems[b % 2])

    o_descs = {}
    d = issue_x(0)
    for b in range(NBLK):
        d_next = issue_x(b + 1) if b + 1 < NBLK else None
        d.wait()
        if b >= 2:
            o_descs[b - 2].wait()
        x_v = x_bufs[b % 2]
        o_v = o_bufs[b % 2]

        def do_rows(i, cs, x_v=x_v, o_v=o_v):
            r0 = i * UNROLL
            for u in range(UNROLL):
                ridx = zeros + (r0 + u)
                for jb in range(OUT_F // 16):
                    g = plsc.load_gather(x_v, [ridx, cs[jb]])
                    o_v[r0 + u, pl.ds(jb * 16, 16)] = g
            return cs

        lax.fori_loop(0, BLK // UNROLL, do_rows, cbase)
        o_descs[b] = issue_o(b)
        d = d_next

    o_descs[NBLK - 2].wait()
    o_descs[NBLK - 1].wait()


@jax.jit
def _sc_gather(x, cols):
    mesh = plsc.VectorSubcoreMesh(core_axis_name="c", subcore_axis_name="s")
    return pl.kernel(
        _sc_body,
        out_type=jax.ShapeDtypeStruct((N_ROWS, OUT_F), jnp.float32),
        mesh=mesh,
        scratch_types=[
            pltpu.VMEM((OUT_F,), jnp.int32),
            pltpu.VMEM((BLK, N_FEATS), jnp.float32),
            pltpu.VMEM((BLK, N_FEATS), jnp.float32),
            pltpu.VMEM((BLK, OUT_F), jnp.float32),
            pltpu.VMEM((BLK, OUT_F), jnp.float32),
            pltpu.SemaphoreType.DMA,
            pltpu.SemaphoreType.DMA,
            pltpu.SemaphoreType.DMA,
            pltpu.SemaphoreType.DMA,
        ],
        compiler_params=pltpu.CompilerParams(needs_layout_passes=False),
    )(x, cols)


def kernel(x, columns):
    return _sc_gather(x, columns.astype(jnp.int32))
